# Initial kernel scaffold; baseline (speedup 1.0000x reference)
import jax
import jax.numpy as jnp
from jax.experimental import pallas as pl


def kernel(bags, W_enc, b_enc, W_att, b_att, bn_gamma, bn_beta, bn_mean, bn_var, W_head, b_head):
    raise NotImplementedError("write your pallas kernel here")


# speedup vs baseline: 1.3349x; 1.3349x over previous
import functools

import jax
import jax.numpy as jnp
from jax.experimental import pallas as pl
from jax.experimental.pallas import tpu as pltpu

_K = 20
_NEG = -3.0e38
_LANES = 128


def _mil_kernel(bags_ref, w_enc_ref, b_enc_ref, w_att_ref, b_att_ref,
                gamma_ref, beta_ref, mean_ref, var_ref, w_head_ref,
                b_head_ref, out_ref, emb_ref, scores_ref, mask_ref,
                *, bb, nb_blk, n_total, k):
    n_i = pl.program_id(1)
    num_n = pl.num_programs(1)

    x = bags_ref[...]
    f = x.shape[-1]
    z = w_enc_ref.shape[-1]
    x2 = x.reshape(bb * nb_blk, f)
    emb = jnp.dot(x2, w_enc_ref[...], preferred_element_type=jnp.float32)
    emb = jnp.maximum(emb + b_enc_ref[...], 0.0)
    s = jnp.dot(emb, w_att_ref[...],
                preferred_element_type=jnp.float32)[:, 0] + b_att_ref[0, 0]
    emb_ref[:, pl.ds(n_i * nb_blk, nb_blk), :] = emb.reshape(bb, nb_blk, z)
    scores_ref[:, pl.ds(n_i * nb_blk, nb_blk)] = s.reshape(bb, nb_blk)

    @pl.when(n_i == num_n - 1)
    def _finalize():
        scores = scores_ref[...]
        iota = jax.lax.broadcasted_iota(jnp.int32, (bb, n_total), 1)
        mask_ref[...] = jnp.zeros((bb, n_total), jnp.float32)

        def body(i, carry):
            vmax, denom = carry
            sel = mask_ref[...]
            cur = jnp.where(sel > 0.0, _NEG, scores)
            m = jnp.max(cur, axis=1, keepdims=True)
            cand = jnp.where(cur == m, iota, n_total)
            idx = jnp.min(cand, axis=1, keepdims=True)
            hit = iota == idx
            mask_ref[...] = jnp.where(hit, 1.0, sel)
            vmax_new = jnp.where(i == 0, m, vmax)
            denom_new = denom + jnp.exp(m - vmax_new)
            return vmax_new, denom_new

        vmax0 = jnp.full((bb, 1), _NEG, jnp.float32)
        den0 = jnp.zeros((bb, 1), jnp.float32)
        vmax, denom = jax.lax.fori_loop(0, k, body, (vmax0, den0))

        sel = mask_ref[...]
        coeff = jnp.where(sel > 0.0, jnp.exp(scores - vmax), 0.0) / denom

        ws = jnp.zeros((bb, z), jnp.float32)
        for c in range(num_n):
            e = emb_ref[:, c * nb_blk:(c + 1) * nb_blk, :]
            cf = coeff[:, c * nb_blk:(c + 1) * nb_blk]
            ws = ws + jnp.sum(e * cf[:, :, None], axis=1)

        bn = (ws - mean_ref[...]) * jax.lax.rsqrt(var_ref[...] + 1e-5)
        bn = bn * gamma_ref[...] + beta_ref[...]
        out = jnp.dot(bn, w_head_ref[...], preferred_element_type=jnp.float32)
        out_ref[...] = out + b_head_ref[...]


def kernel(bags, W_enc, b_enc, W_att, b_att, bn_gamma, bn_beta, bn_mean,
           bn_var, W_head, b_head):
    B, N, F = bags.shape
    Z = W_enc.shape[1]
    NOUT = W_head.shape[1]
    k = min(_K, N)

    BB = 8 if B % 8 == 0 else B
    NB = 1024 if N % 1024 == 0 else N
    num_n = N // NB

    b_enc2 = b_enc.reshape(1, Z)
    w_att2 = jnp.zeros((Z, _LANES), jnp.float32).at[:, 0:1].set(W_att)
    b_att2 = b_att.reshape(1, 1)
    gamma2 = bn_gamma.reshape(1, Z)
    beta2 = bn_beta.reshape(1, Z)
    mean2 = bn_mean.reshape(1, Z)
    var2 = bn_var.reshape(1, Z)
    w_head_p = jnp.zeros((Z, _LANES), jnp.float32).at[:, :NOUT].set(W_head)
    b_head_p = jnp.zeros((1, _LANES), jnp.float32).at[:, :NOUT].set(b_head)

    body = functools.partial(_mil_kernel, bb=BB, nb_blk=NB, n_total=N, k=k)

    out = pl.pallas_call(
        body,
        grid=(B // BB, num_n),
        in_specs=[
            pl.BlockSpec((BB, NB, F), lambda b, n: (b, n, 0)),
            pl.BlockSpec((F, Z), lambda b, n: (0, 0)),
            pl.BlockSpec((1, Z), lambda b, n: (0, 0)),
            pl.BlockSpec((Z, _LANES), lambda b, n: (0, 0)),
            pl.BlockSpec((1, 1), lambda b, n: (0, 0)),
            pl.BlockSpec((1, Z), lambda b, n: (0, 0)),
            pl.BlockSpec((1, Z), lambda b, n: (0, 0)),
            pl.BlockSpec((1, Z), lambda b, n: (0, 0)),
            pl.BlockSpec((1, Z), lambda b, n: (0, 0)),
            pl.BlockSpec((Z, _LANES), lambda b, n: (0, 0)),
            pl.BlockSpec((1, _LANES), lambda b, n: (0, 0)),
        ],
        out_specs=pl.BlockSpec((BB, _LANES), lambda b, n: (b, 0)),
        out_shape=jax.ShapeDtypeStruct((B, _LANES), jnp.float32),
        scratch_shapes=[
            pltpu.VMEM((BB, N, Z), jnp.float32),
            pltpu.VMEM((BB, N), jnp.float32),
            pltpu.VMEM((BB, N), jnp.float32),
        ],
        compiler_params=pltpu.CompilerParams(
            vmem_limit_bytes=100 * 1024 * 1024),
    )(bags, W_enc, b_enc2, w_att2, b_att2, gamma2, beta2, mean2, var2,
      w_head_p, b_head_p)
    return out[:, :NOUT]


# speedup vs baseline: 1.8342x; 1.3740x over previous
import functools

import jax
import jax.numpy as jnp
from jax.experimental import pallas as pl
from jax.experimental.pallas import tpu as pltpu

_K = 20
_NEG = -3.0e38
_LANES = 128


def _mil_kernel(bags_ref, w_enc_ref, b_enc_ref, w_att_ref, b_att_ref,
                gamma_ref, beta_ref, mean_ref, var_ref, w_head_ref,
                b_head_ref, out_ref, emb_ref, scores_ref,
                *, bb, nb_blk, n_total, k):
    n_i = pl.program_id(1)
    num_n = pl.num_programs(1)

    x = bags_ref[...]
    f = x.shape[-1]
    z = w_enc_ref.shape[-1]
    x2 = x.reshape(bb * nb_blk, f)
    emb = jnp.dot(x2, w_enc_ref[...], preferred_element_type=jnp.float32)
    emb = jnp.maximum(emb + b_enc_ref[...], 0.0)
    s = jnp.dot(emb, w_att_ref[...],
                preferred_element_type=jnp.float32)[:, 0] + b_att_ref[0, 0]
    emb_ref[:, pl.ds(n_i * nb_blk, nb_blk), :] = emb.reshape(bb, nb_blk, z)
    scores_ref[:, pl.ds(n_i * nb_blk, nb_blk)] = s.reshape(bb, nb_blk)

    @pl.when(n_i == num_n - 1)
    def _finalize():
        iota = jax.lax.broadcasted_iota(jnp.int32, (bb, n_total), 1)

        def body(i, carry):
            cur, vmax, denom, acc = carry
            m = jnp.max(cur, axis=1, keepdims=True)
            cand = jnp.where(cur == m, iota, n_total)
            idx = jnp.min(cand, axis=1, keepdims=True)
            cur = jnp.where(iota == idx, _NEG, cur)
            rows = []
            for b in range(bb):
                i_b = idx[b, 0]
                rows.append(emb_ref[b, pl.ds(i_b, 1), :])
            sel_rows = jnp.concatenate(rows, axis=0)
            vmax_new = jnp.where(i == 0, m, vmax)
            wexp = jnp.exp(m - vmax_new)
            return cur, vmax_new, denom + wexp, acc + wexp * sel_rows

        vmax0 = jnp.full((bb, 1), _NEG, jnp.float32)
        den0 = jnp.zeros((bb, 1), jnp.float32)
        acc0 = jnp.zeros((bb, z), jnp.float32)
        _, _, denom, acc = jax.lax.fori_loop(
            0, k, body, (scores_ref[...], vmax0, den0, acc0))
        ws = acc / denom

        bn = (ws - mean_ref[...]) * jax.lax.rsqrt(var_ref[...] + 1e-5)
        bn = bn * gamma_ref[...] + beta_ref[...]
        out = jnp.dot(bn, w_head_ref[...], preferred_element_type=jnp.float32)
        out_ref[...] = out + b_head_ref[...]


def kernel(bags, W_enc, b_enc, W_att, b_att, bn_gamma, bn_beta, bn_mean,
           bn_var, W_head, b_head):
    B, N, F = bags.shape
    Z = W_enc.shape[1]
    NOUT = W_head.shape[1]
    k = min(_K, N)

    BB = 8 if B % 8 == 0 else B
    NB = 1024 if N % 1024 == 0 else N
    num_n = N // NB

    b_enc2 = b_enc.reshape(1, Z)
    w_att2 = jnp.zeros((Z, _LANES), jnp.float32).at[:, 0:1].set(W_att)
    b_att2 = b_att.reshape(1, 1)
    gamma2 = bn_gamma.reshape(1, Z)
    beta2 = bn_beta.reshape(1, Z)
    mean2 = bn_mean.reshape(1, Z)
    var2 = bn_var.reshape(1, Z)
    w_head_p = jnp.zeros((Z, _LANES), jnp.float32).at[:, :NOUT].set(W_head)
    b_head_p = jnp.zeros((1, _LANES), jnp.float32).at[:, :NOUT].set(b_head)

    body = functools.partial(_mil_kernel, bb=BB, nb_blk=NB, n_total=N, k=k)

    out = pl.pallas_call(
        body,
        grid=(B // BB, num_n),
        in_specs=[
            pl.BlockSpec((BB, NB, F), lambda b, n: (b, n, 0)),
            pl.BlockSpec((F, Z), lambda b, n: (0, 0)),
            pl.BlockSpec((1, Z), lambda b, n: (0, 0)),
            pl.BlockSpec((Z, _LANES), lambda b, n: (0, 0)),
            pl.BlockSpec((1, 1), lambda b, n: (0, 0)),
            pl.BlockSpec((1, Z), lambda b, n: (0, 0)),
            pl.BlockSpec((1, Z), lambda b, n: (0, 0)),
            pl.BlockSpec((1, Z), lambda b, n: (0, 0)),
            pl.BlockSpec((1, Z), lambda b, n: (0, 0)),
            pl.BlockSpec((Z, _LANES), lambda b, n: (0, 0)),
            pl.BlockSpec((1, _LANES), lambda b, n: (0, 0)),
        ],
        out_specs=pl.BlockSpec((BB, _LANES), lambda b, n: (b, 0)),
        out_shape=jax.ShapeDtypeStruct((B, _LANES), jnp.float32),
        scratch_shapes=[
            pltpu.VMEM((BB, N, Z), jnp.float32),
            pltpu.VMEM((BB, N), jnp.float32),
        ],
        compiler_params=pltpu.CompilerParams(
            vmem_limit_bytes=100 * 1024 * 1024),
    )(bags, W_enc, b_enc2, w_att2, b_att2, gamma2, beta2, mean2, var2,
      w_head_p, b_head_p)
    return out[:, :NOUT]


# speedup vs baseline: 2.0479x; 1.1165x over previous
import functools

import jax
import jax.numpy as jnp
from jax.experimental import pallas as pl
from jax.experimental.pallas import tpu as pltpu

_K = 20
_NEG = -3.0e38
_LANES = 128


def _mil_kernel(bags_ref, w_enc_ref, b_enc_ref, w_att_ref, b_att_ref,
                gamma_ref, beta_ref, mean_ref, var_ref, w_head_ref,
                b_head_ref, out_ref, emb_ref, scores_ref,
                *, bb, nb_blk, n_total, k):
    n_i = pl.program_id(1)
    num_n = pl.num_programs(1)

    x = bags_ref[...]
    f = x.shape[-1]
    z = w_enc_ref.shape[-1]
    x2 = x.reshape(bb * nb_blk, f)
    emb = jnp.dot(x2, w_enc_ref[...], preferred_element_type=jnp.float32)
    emb = jnp.maximum(emb + b_enc_ref[...], 0.0)
    s = jnp.dot(emb, w_att_ref[...],
                preferred_element_type=jnp.float32)[:, 0] + b_att_ref[0, 0]
    emb_ref[:, pl.ds(n_i * nb_blk, nb_blk), :] = emb.reshape(bb, nb_blk, z)
    scores_ref[:, pl.ds(n_i * nb_blk, nb_blk)] = s.reshape(bb, nb_blk)

    @pl.when(n_i == num_n - 1)
    def _finalize():
        iota = jax.lax.broadcasted_iota(jnp.int32, (bb, n_total), 1)

        def body(i, carry):
            cur, vmax, denom, acc = carry
            m = jnp.max(cur, axis=1, keepdims=True)
            cand = jnp.where(cur == m, iota, n_total)
            idx = jnp.min(cand, axis=1, keepdims=True)
            cur = jnp.where(iota == idx, _NEG, cur)
            rows = []
            for b in range(bb):
                i_b = idx[b, 0]
                rows.append(emb_ref[b, pl.ds(i_b, 1), :])
            sel_rows = jnp.concatenate(rows, axis=0)
            vmax_new = jnp.where(i == 0, m, vmax)
            wexp = jnp.exp(m - vmax_new)
            return cur, vmax_new, denom + wexp, acc + wexp * sel_rows

        vmax0 = jnp.full((bb, 1), _NEG, jnp.float32)
        den0 = jnp.zeros((bb, 1), jnp.float32)
        acc0 = jnp.zeros((bb, z), jnp.float32)
        _, _, denom, acc = jax.lax.fori_loop(
            0, k, body, (scores_ref[...], vmax0, den0, acc0))
        ws = acc / denom

        bn = (ws - mean_ref[...]) * jax.lax.rsqrt(var_ref[...] + 1e-5)
        bn = bn * gamma_ref[...] + beta_ref[...]
        out = jnp.dot(bn, w_head_ref[...], preferred_element_type=jnp.float32)
        out_ref[...] = out + b_head_ref[...]


def kernel(bags, W_enc, b_enc, W_att, b_att, bn_gamma, bn_beta, bn_mean,
           bn_var, W_head, b_head):
    B, N, F = bags.shape
    Z = W_enc.shape[1]
    NOUT = W_head.shape[1]
    k = min(_K, N)

    BB = 8 if B % 8 == 0 else B
    NB = 2048 if N % 2048 == 0 else N
    num_n = N // NB

    b_enc2 = b_enc.reshape(1, Z)
    w_att2 = jnp.zeros((Z, _LANES), jnp.float32).at[:, 0:1].set(W_att)
    b_att2 = b_att.reshape(1, 1)
    gamma2 = bn_gamma.reshape(1, Z)
    beta2 = bn_beta.reshape(1, Z)
    mean2 = bn_mean.reshape(1, Z)
    var2 = bn_var.reshape(1, Z)
    w_head_p = jnp.zeros((Z, _LANES), jnp.float32).at[:, :NOUT].set(W_head)
    b_head_p = jnp.zeros((1, _LANES), jnp.float32).at[:, :NOUT].set(b_head)

    body = functools.partial(_mil_kernel, bb=BB, nb_blk=NB, n_total=N, k=k)

    out = pl.pallas_call(
        body,
        grid=(B // BB, num_n),
        in_specs=[
            pl.BlockSpec((BB, NB, F), lambda b, n: (b, n, 0)),
            pl.BlockSpec((F, Z), lambda b, n: (0, 0)),
            pl.BlockSpec((1, Z), lambda b, n: (0, 0)),
            pl.BlockSpec((Z, _LANES), lambda b, n: (0, 0)),
            pl.BlockSpec((1, 1), lambda b, n: (0, 0)),
            pl.BlockSpec((1, Z), lambda b, n: (0, 0)),
            pl.BlockSpec((1, Z), lambda b, n: (0, 0)),
            pl.BlockSpec((1, Z), lambda b, n: (0, 0)),
            pl.BlockSpec((1, Z), lambda b, n: (0, 0)),
            pl.BlockSpec((Z, _LANES), lambda b, n: (0, 0)),
            pl.BlockSpec((1, _LANES), lambda b, n: (0, 0)),
        ],
        out_specs=pl.BlockSpec((BB, _LANES), lambda b, n: (b, 0)),
        out_shape=jax.ShapeDtypeStruct((B, _LANES), jnp.float32),
        scratch_shapes=[
            pltpu.VMEM((BB, N, Z), jnp.float32),
            pltpu.VMEM((BB, N), jnp.float32),
        ],
        compiler_params=pltpu.CompilerParams(
            vmem_limit_bytes=100 * 1024 * 1024),
    )(bags, W_enc, b_enc2, w_att2, b_att2, gamma2, beta2, mean2, var2,
      w_head_p, b_head_p)
    return out[:, :NOUT]


# speedup vs baseline: 2.0481x; 1.0001x over previous
import functools

import jax
import jax.numpy as jnp
from jax.experimental import pallas as pl
from jax.experimental.pallas import tpu as pltpu

_K = 20
_NEG = -3.0e38
_LANES = 128


def _mil_kernel(bags_ref, w_enc_ref, b_enc_ref, w_att_ref, b_att_ref,
                gamma_ref, beta_ref, mean_ref, var_ref, w_head_ref,
                b_head_ref, out_ref, emb_ref, scores_ref,
                *, bb, nb_blk, n_total, k):
    n_i = pl.program_id(1)
    num_n = pl.num_programs(1)

    x = bags_ref[...]
    f = x.shape[-1]
    z = w_enc_ref.shape[-1]
    x2 = x.reshape(bb * nb_blk, f)
    emb = jnp.dot(x2, w_enc_ref[...], preferred_element_type=jnp.float32)
    emb = jnp.maximum(emb + b_enc_ref[...], 0.0)
    s = jnp.dot(emb, w_att_ref[...],
                preferred_element_type=jnp.float32)[:, 0] + b_att_ref[0, 0]
    emb_ref[:, pl.ds(n_i * nb_blk, nb_blk), :] = emb.reshape(bb, nb_blk, z)
    scores_ref[:, pl.ds(n_i * nb_blk, nb_blk)] = s.reshape(bb, nb_blk)

    @pl.when(n_i == num_n - 1)
    def _finalize():
        iota = jax.lax.broadcasted_iota(jnp.int32, (bb, n_total), 1)

        def body(i, carry):
            vmax, denom, acc = carry
            cur = scores_ref[...]
            m = jnp.max(cur, axis=1, keepdims=True)
            cand = jnp.where(cur == m, iota, n_total)
            idx = jnp.min(cand, axis=1, keepdims=True)
            scores_ref[...] = jnp.where(iota == idx, _NEG, cur)
            rows = []
            for b in range(bb):
                i_b = idx[b, 0]
                rows.append(emb_ref[b, pl.ds(i_b, 1), :])
            sel_rows = jnp.concatenate(rows, axis=0)
            vmax_new = jnp.where(i == 0, m, vmax)
            wexp = jnp.exp(m - vmax_new)
            return vmax_new, denom + wexp, acc + wexp * sel_rows

        vmax0 = jnp.full((bb, 1), _NEG, jnp.float32)
        den0 = jnp.zeros((bb, 1), jnp.float32)
        acc0 = jnp.zeros((bb, z), jnp.float32)
        _, denom, acc = jax.lax.fori_loop(0, k, body, (vmax0, den0, acc0))
        ws = acc / denom

        bn = (ws - mean_ref[...]) * jax.lax.rsqrt(var_ref[...] + 1e-5)
        bn = bn * gamma_ref[...] + beta_ref[...]
        out = jnp.dot(bn, w_head_ref[...], preferred_element_type=jnp.float32)
        out_ref[...] = out + b_head_ref[...]


def kernel(bags, W_enc, b_enc, W_att, b_att, bn_gamma, bn_beta, bn_mean,
           bn_var, W_head, b_head):
    B, N, F = bags.shape
    Z = W_enc.shape[1]
    NOUT = W_head.shape[1]
    k = min(_K, N)

    BB = 8 if B % 8 == 0 else B
    NB = 2048 if N % 2048 == 0 else N
    num_n = N // NB

    b_enc2 = b_enc.reshape(1, Z)
    w_att2 = jnp.zeros((Z, _LANES), jnp.float32).at[:, 0:1].set(W_att)
    b_att2 = b_att.reshape(1, 1)
    gamma2 = bn_gamma.reshape(1, Z)
    beta2 = bn_beta.reshape(1, Z)
    mean2 = bn_mean.reshape(1, Z)
    var2 = bn_var.reshape(1, Z)
    w_head_p = jnp.zeros((Z, _LANES), jnp.float32).at[:, :NOUT].set(W_head)
    b_head_p = jnp.zeros((1, _LANES), jnp.float32).at[:, :NOUT].set(b_head)

    body = functools.partial(_mil_kernel, bb=BB, nb_blk=NB, n_total=N, k=k)

    out = pl.pallas_call(
        body,
        grid=(B // BB, num_n),
        in_specs=[
            pl.BlockSpec((BB, NB, F), lambda b, n: (b, n, 0)),
            pl.BlockSpec((F, Z), lambda b, n: (0, 0)),
            pl.BlockSpec((1, Z), lambda b, n: (0, 0)),
            pl.BlockSpec((Z, _LANES), lambda b, n: (0, 0)),
            pl.BlockSpec((1, 1), lambda b, n: (0, 0)),
            pl.BlockSpec((1, Z), lambda b, n: (0, 0)),
            pl.BlockSpec((1, Z), lambda b, n: (0, 0)),
            pl.BlockSpec((1, Z), lambda b, n: (0, 0)),
            pl.BlockSpec((1, Z), lambda b, n: (0, 0)),
            pl.BlockSpec((Z, _LANES), lambda b, n: (0, 0)),
            pl.BlockSpec((1, _LANES), lambda b, n: (0, 0)),
        ],
        out_specs=pl.BlockSpec((BB, _LANES), lambda b, n: (b, 0)),
        out_shape=jax.ShapeDtypeStruct((B, _LANES), jnp.float32),
        scratch_shapes=[
            pltpu.VMEM((BB, N, Z), jnp.float32),
            pltpu.VMEM((BB, N), jnp.float32),
        ],
        compiler_params=pltpu.CompilerParams(
            vmem_limit_bytes=100 * 1024 * 1024),
    )(bags, W_enc, b_enc2, w_att2, b_att2, gamma2, beta2, mean2, var2,
      w_head_p, b_head_p)
    return out[:, :NOUT]
